# baseline (speedup 1.0000x reference)
import functools

import jax
import jax.numpy as jnp
from jax import lax
from jax.experimental import pallas as pl
from jax.experimental.pallas import tpu as pltpu
from jax.experimental.pallas import tpu_sc as plsc


def _build_gather(B: int, D: int, V2: int):
    info = plsc.get_sparse_core_info()
    NC, NS, L = info.num_cores, info.num_subcores, info.num_lanes
    NW = NC * NS
    assert B % (8 * NW) == 0 and D % L == 0
    b_per_w = B // NW
    CHUNK = 256
    assert b_per_w % CHUNK == 0
    NCHUNK = b_per_w // CHUNK
    G = CHUNK // L

    mesh = plsc.VectorSubcoreMesh(core_axis_name="c", subcore_axis_name="s")

    @functools.partial(
        pl.kernel,
        mesh=mesh,
        compiler_params=pltpu.CompilerParams(
            use_tc_tiling_on_sc=True, needs_layout_passes=False
        ),
        out_type=jax.ShapeDtypeStruct((B, D), jnp.float32),
        scratch_types=[
            pltpu.VMEM((b_per_w,), jnp.int32),
            pltpu.VMEM((CHUNK, 2 * D), jnp.float32),
            pltpu.VMEM((CHUNK, D), jnp.float32),
            pltpu.SemaphoreType.DMA,
        ],
    )
    def gather2x(t128_hbm, idx_hbm, out_hbm, idx_v, rows_v, out_v, sem):
        wid = lax.axis_index("s") * NC + lax.axis_index("c")
        base = wid * b_per_w
        pltpu.sync_copy(idx_hbm.at[pl.ds(base, b_per_w)], idx_v)
        lanes = lax.iota(jnp.int32, L)

        def chunk_body(j, carry):
            cb = j * CHUNK

            def fire(g, c2):
                vec = idx_v[pl.ds(cb + g * L, L)]
                pltpu.async_copy(
                    t128_hbm.at[vec >> 1], rows_v.at[pl.ds(g * L, L)], sem
                )
                return c2

            lax.fori_loop(0, G, fire, 0)

            def drain(g, c2):
                pltpu.make_async_copy(
                    t128_hbm.at[pl.ds(0, L)], rows_v.at[pl.ds(g * L, L)], sem
                ).wait()
                return c2

            lax.fori_loop(0, G, drain, 0)

            def compact(g, c2):
                vec = idx_v[pl.ds(cb + g * L, L)]
                half = (vec & 1) * D
                row = g * L + lanes
                for c in range(D):
                    vals = plsc.load_gather(rows_v, [row, half + c])
                    plsc.store_scatter(out_v, [row, jnp.full((L,), c, jnp.int32)], vals * 2.0)
                return c2

            lax.fori_loop(0, G, compact, 0)
            pltpu.sync_copy(out_v, out_hbm.at[pl.ds(base + cb, CHUNK)])
            return carry

        lax.fori_loop(0, NCHUNK, chunk_body, 0)

    return gather2x


def kernel(inputs, token_table, position_table):
    del position_table
    Bx, S = inputs.shape
    V, D = token_table.shape
    idx = inputs.reshape(-1).astype(jnp.int32)
    t128 = token_table.reshape(V // 2, 2 * D)
    out = _build_gather(Bx * S, D, V // 2)(t128, idx)
    return out.reshape(Bx, S, D)


# speedup vs baseline: 2.4084x; 2.4084x over previous
import functools

import jax
import jax.numpy as jnp
from jax import lax
from jax.experimental import pallas as pl
from jax.experimental.pallas import tpu as pltpu
from jax.experimental.pallas import tpu_sc as plsc


def _build_gather(B: int, D: int):
    info = plsc.get_sparse_core_info()
    NC, NS, L = info.num_cores, info.num_subcores, info.num_lanes
    NW = NC * NS
    assert B % (8 * NW) == 0 and D % L == 0
    b_per_w = B // NW
    CHUNK = 640
    assert b_per_w % CHUNK == 0
    NCHUNK = b_per_w // CHUNK

    mesh = plsc.VectorSubcoreMesh(core_axis_name="c", subcore_axis_name="s")

    @functools.partial(
        pl.kernel,
        mesh=mesh,
        compiler_params=pltpu.CompilerParams(
            use_tc_tiling_on_sc=True, needs_layout_passes=False
        ),
        out_type=jax.ShapeDtypeStruct((B, D), jnp.float32),
        scratch_types=[
            pltpu.SMEM((CHUNK,), jnp.int32),
            pltpu.VMEM((CHUNK, D), jnp.float32),
            pltpu.VMEM_SHARED((16, CHUNK), jnp.int32),
            pltpu.SemaphoreType.DMA,
            pltpu.SemaphoreType.DMA,
        ],
    )
    def gather2x(table_hbm, idx_hbm, out_hbm, idx_s, rows_v, idx_sh, sem, sem2):
        wid = lax.axis_index("s") * NC + lax.axis_index("c")
        base = wid * b_per_w

        def chunk_body(j, carry):
            cb = j * CHUNK
            sid = lax.axis_index("s")
            pltpu.sync_copy(idx_hbm.at[pl.ds(base + cb, CHUNK)], idx_sh.at[sid])
            pltpu.sync_copy(idx_sh.at[sid], idx_s)

            def fire(r, c2):
                row = idx_s[r]
                pltpu.async_copy(
                    table_hbm.at[pl.ds(row, 1)], rows_v.at[pl.ds(r, 1)], sem
                )
                return c2

            lax.fori_loop(0, CHUNK, fire, 0)

            def drain(r, c2):
                pltpu.make_async_copy(
                    table_hbm.at[pl.ds(0, 1)], rows_v.at[pl.ds(r, 1)], sem
                ).wait()
                return c2

            lax.fori_loop(0, CHUNK, drain, 0)

            def mul_body(r, c2):
                for c in range(D // L):
                    sl = pl.ds(c * L, L)
                    rows_v[r, sl] = rows_v[r, sl] + rows_v[r, sl]
                return c2

            lax.fori_loop(0, CHUNK, mul_body, 0, unroll=4)
            pltpu.sync_copy(rows_v, out_hbm.at[pl.ds(base + cb, CHUNK)])
            return carry

        lax.fori_loop(0, NCHUNK, chunk_body, 0)

    return gather2x


def kernel(inputs, token_table, position_table):
    del position_table
    Bx, S = inputs.shape
    V, D = token_table.shape
    idx = inputs.reshape(-1).astype(jnp.int32)
    out = _build_gather(Bx * S, D)(token_table, idx)
    return out.reshape(Bx, S, D)
